# baseline (speedup 1.0000x reference)
import functools

import jax
import jax.numpy as jnp
from jax import lax
from jax.experimental import pallas as pl
from jax.experimental.pallas import tpu as pltpu
from jax.experimental.pallas import tpu_sc as plsc

N, D = 16384, 2048
NC, NS = 2, 16
NW = NC * NS
ROWS_W = N // NW
CHUNK = 16
NCHUNK = ROWS_W // CHUNK
GPC = CHUNK // 4


def _sc_body(x_hbm, o_hbm, buf0, buf1, sin0, sin1):
    wid = lax.axis_index("s") * NC + lax.axis_index("c")
    base = wid * ROWS_W

    bufs = (buf0, buf1)
    sins = (sin0, sin1)

    zero = jnp.zeros((16,), jnp.float32)

    def issue_gather(k, b):
        pltpu.async_copy(
            x_hbm.at[pl.ds(base + k * CHUNK, CHUNK)], bufs[b], sins[b]
        )

    def wait_gather(k, b):
        pltpu.make_async_copy(
            x_hbm.at[pl.ds(base + k * CHUNK, CHUNK)], bufs[b], sins[b]
        ).wait()

    issue_gather(0, 0)
    issue_gather(1, 1)

    def step(t, c):
        for b in range(2):
            k = 2 * t + b
            wait_gather(k, b)

            def zrow(j, cc):
                for g in range(GPC):
                    bufs[b][4 * g, pl.ds(j * 16, 16)] = zero
                return cc

            lax.fori_loop(0, D // 16, zrow, 0)
            pltpu.sync_copy(bufs[b], o_hbm.at[pl.ds(base + k * CHUNK, CHUNK)])

            @pl.when(k + 2 < NCHUNK)
            def _():
                issue_gather(k + 2, b)

        return c

    lax.fori_loop(0, NCHUNK // 2, step, 0)


_sc_kernel = functools.partial(
    pl.kernel,
    mesh=plsc.VectorSubcoreMesh(core_axis_name="c", subcore_axis_name="s"),
    out_type=jax.ShapeDtypeStruct((N, D), jnp.float32),
    scratch_types=[
        pltpu.VMEM((CHUNK, D), jnp.float32),
        pltpu.VMEM((CHUNK, D), jnp.float32),
        pltpu.SemaphoreType.DMA,
        pltpu.SemaphoreType.DMA,
    ],
)(_sc_body)


def kernel(x):
    return _sc_kernel(x)


# speedup vs baseline: 1.0102x; 1.0102x over previous
import functools

import jax
import jax.numpy as jnp
from jax import lax
from jax.experimental import pallas as pl
from jax.experimental.pallas import tpu as pltpu
from jax.experimental.pallas import tpu_sc as plsc

N, D = 16384, 2048
NC, NS = 2, 16
NW = NC * NS
ROWS_W = N // NW
CHUNK = 16
NCHUNK = ROWS_W // CHUNK
GPC = CHUNK // 4
NBUF = 3


def _sc_body(x_hbm, o_hbm, buf0, buf1, buf2, sin0, sin1, sin2, sout0, sout1, sout2):
    wid = lax.axis_index("s") * NC + lax.axis_index("c")
    base = wid * ROWS_W

    bufs = (buf0, buf1, buf2)
    sins = (sin0, sin1, sin2)
    souts = (sout0, sout1, sout2)

    zero = jnp.zeros((16,), jnp.float32)

    def gather(k, b):
        return pltpu.make_async_copy(
            x_hbm.at[pl.ds(base + k * CHUNK, CHUNK)], bufs[b], sins[b]
        )

    def write(k, b):
        return pltpu.make_async_copy(
            bufs[b], o_hbm.at[pl.ds(base + k * CHUNK, CHUNK)], souts[b]
        )

    gather(0, 0).start()
    gather(1, 1).start()

    def step(k, c):
        for b in range(NBUF):

            @pl.when(k % NBUF == b)
            def _():
                gather(k, b).wait()

                def zrow(j, cc):
                    for g in range(GPC):
                        bufs[b][4 * g, pl.ds(j * 16, 16)] = zero
                    return cc

                lax.fori_loop(0, D // 16, zrow, 0)
                write(k, b).start()

                @pl.when(k + 2 < NCHUNK)
                def _():
                    bn = (b + 2) % NBUF

                    @pl.when(k >= 1)
                    def _():
                        write(k - 1, bn).wait()

                    gather(k + 2, bn).start()

        return c

    lax.fori_loop(0, NCHUNK, step, 0)

    for k in (NCHUNK - 3, NCHUNK - 2, NCHUNK - 1):
        write(k, k % NBUF).wait()


_sc_kernel = functools.partial(
    pl.kernel,
    mesh=plsc.VectorSubcoreMesh(core_axis_name="c", subcore_axis_name="s"),
    out_type=jax.ShapeDtypeStruct((N, D), jnp.float32),
    scratch_types=[
        pltpu.VMEM((CHUNK, D), jnp.float32),
        pltpu.VMEM((CHUNK, D), jnp.float32),
        pltpu.VMEM((CHUNK, D), jnp.float32),
        pltpu.SemaphoreType.DMA,
        pltpu.SemaphoreType.DMA,
        pltpu.SemaphoreType.DMA,
        pltpu.SemaphoreType.DMA,
        pltpu.SemaphoreType.DMA,
        pltpu.SemaphoreType.DMA,
    ],
)(_sc_body)


def kernel(x):
    return _sc_kernel(x)


# speedup vs baseline: 1.0122x; 1.0020x over previous
import functools

import jax
import jax.numpy as jnp
from jax import lax
from jax.experimental import pallas as pl
from jax.experimental.pallas import tpu as pltpu
from jax.experimental.pallas import tpu_sc as plsc

N, D = 16384, 2048
NC, NS = 2, 16
NW = NC * NS
ROWS_W = N // NW
CHUNK = 16
NCHUNK = ROWS_W // CHUNK
GPC = CHUNK // 4
NBUF = 3


def _sc_body(x_hbm, o_hbm, buf0, buf1, buf2, sin0, sin1, sin2, sout0, sout1, sout2):
    wid = lax.axis_index("s") * NC + lax.axis_index("c")
    base = wid * ROWS_W

    bufs = (buf0, buf1, buf2)
    sins = (sin0, sin1, sin2)
    souts = (sout0, sout1, sout2)

    zero = jnp.zeros((16,), jnp.float32)

    def gather(k, b):
        return pltpu.make_async_copy(
            x_hbm.at[pl.ds(base + k * CHUNK, CHUNK)], bufs[b], sins[b]
        )

    def write(k, b):
        return pltpu.make_async_copy(
            bufs[b], o_hbm.at[pl.ds(base + k * CHUNK, CHUNK)], souts[b]
        )

    gather(0, 0).start()
    gather(1, 1).start()

    def step(k, c):
        for b in range(NBUF):

            @pl.when(k % NBUF == b)
            def _():
                gather(k, b).wait()

                def zrow(j, cc):
                    for g in range(GPC):
                        for u in range(4):
                            bufs[b][4 * g, pl.ds((4 * j + u) * 16, 16)] = zero
                    return cc

                lax.fori_loop(0, D // 64, zrow, 0)
                write(k, b).start()

                @pl.when(k + 2 < NCHUNK)
                def _():
                    bn = (b + 2) % NBUF

                    @pl.when(k >= 1)
                    def _():
                        write(k - 1, bn).wait()

                    gather(k + 2, bn).start()

        return c

    lax.fori_loop(0, NCHUNK, step, 0)

    for k in (NCHUNK - 3, NCHUNK - 2, NCHUNK - 1):
        write(k, k % NBUF).wait()


_sc_kernel = functools.partial(
    pl.kernel,
    mesh=plsc.VectorSubcoreMesh(core_axis_name="c", subcore_axis_name="s"),
    out_type=jax.ShapeDtypeStruct((N, D), jnp.float32),
    scratch_types=[
        pltpu.VMEM((CHUNK, D), jnp.float32),
        pltpu.VMEM((CHUNK, D), jnp.float32),
        pltpu.VMEM((CHUNK, D), jnp.float32),
        pltpu.SemaphoreType.DMA,
        pltpu.SemaphoreType.DMA,
        pltpu.SemaphoreType.DMA,
        pltpu.SemaphoreType.DMA,
        pltpu.SemaphoreType.DMA,
        pltpu.SemaphoreType.DMA,
    ],
)(_sc_body)


def kernel(x):
    return _sc_kernel(x)


# speedup vs baseline: 1.0262x; 1.0138x over previous
import functools

import jax
import jax.numpy as jnp
from jax import lax
from jax.experimental import pallas as pl
from jax.experimental.pallas import tpu as pltpu
from jax.experimental.pallas import tpu_sc as plsc

N, D = 16384, 2048
NC, NS = 2, 16
NW = NC * NS
ROWS_W = N // NW
CHUNK = 8
NCHUNK = ROWS_W // CHUNK
GPC = CHUNK // 4
NBUF = 6
AHEAD = 3


def _sc_body(x_hbm, o_hbm, *refs):
    bufs = refs[:NBUF]
    sins = refs[NBUF:2 * NBUF]
    souts = refs[2 * NBUF:3 * NBUF]

    wid = lax.axis_index("s") * NC + lax.axis_index("c")
    base = wid * ROWS_W

    zero = jnp.zeros((16,), jnp.float32)

    def gather(k, b):
        return pltpu.make_async_copy(
            x_hbm.at[pl.ds(base + k * CHUNK, CHUNK)], bufs[b], sins[b]
        )

    def write(k, b):
        return pltpu.make_async_copy(
            bufs[b], o_hbm.at[pl.ds(base + k * CHUNK, CHUNK)], souts[b]
        )

    for k0 in range(AHEAD):
        gather(k0, k0).start()

    def step(k, c):
        for b in range(NBUF):

            @pl.when(k % NBUF == b)
            def _():
                gather(k, b).wait()

                def zrow(j, cc):
                    for g in range(GPC):
                        for u in range(4):
                            bufs[b][4 * g, pl.ds((4 * j + u) * 16, 16)] = zero
                    return cc

                lax.fori_loop(0, D // 64, zrow, 0)
                write(k, b).start()

                @pl.when(k + AHEAD < NCHUNK)
                def _():
                    bn = (b + AHEAD) % NBUF

                    @pl.when(k >= NBUF - AHEAD)
                    def _():
                        write(k + AHEAD - NBUF, bn).wait()

                    gather(k + AHEAD, bn).start()

        return c

    lax.fori_loop(0, NCHUNK, step, 0)

    for k in range(max(0, NCHUNK - NBUF), NCHUNK):
        write(k, k % NBUF).wait()


_sc_kernel = functools.partial(
    pl.kernel,
    mesh=plsc.VectorSubcoreMesh(core_axis_name="c", subcore_axis_name="s"),
    out_type=jax.ShapeDtypeStruct((N, D), jnp.float32),
    scratch_types=(
        [pltpu.VMEM((CHUNK, D), jnp.float32)] * NBUF
        + [pltpu.SemaphoreType.DMA] * (2 * NBUF)
    ),
)(_sc_body)


def kernel(x):
    return _sc_kernel(x)
